# baseline (speedup 1.0000x reference)
import functools

import jax
import jax.numpy as jnp
from jax import lax
from jax.experimental import pallas as pl
from jax.experimental.pallas import tpu as pltpu
from jax.experimental.pallas import tpu_sc as plsc


def _sc_gather_sum(idx_flat, pos, neg, table, B, L, NEGK, D):
    info = plsc.get_sparse_core_info()
    NC, NS = info.num_cores, info.num_subcores
    NW = NC * NS
    BPT = B // NW
    G = 2
    NG = BPT // G
    CHUNK = 80
    NCH = (G * L) // CHUNK
    NV = D // 16

    mesh = plsc.VectorSubcoreMesh(core_axis_name="c", subcore_axis_name="s")

    @functools.partial(
        pl.kernel,
        mesh=mesh,
        compiler_params=pltpu.CompilerParams(use_tc_tiling_on_sc=False),
        out_type=(
            jax.ShapeDtypeStruct((B, D), jnp.float32),
            jax.ShapeDtypeStruct((B, D), jnp.float32),
            jax.ShapeDtypeStruct((B * NEGK, D), jnp.float32),
        ),
        scratch_types=[
            pltpu.VMEM((G * L,), jnp.int32),
            pltpu.VMEM((G * L, D), jnp.float32),
            pltpu.VMEM((BPT, D), jnp.float32),
            pltpu.VMEM((BPT,), jnp.int32),
            pltpu.VMEM((BPT, D), jnp.float32),
            pltpu.VMEM((BPT * NEGK,), jnp.int32),
            pltpu.VMEM((BPT * NEGK, D), jnp.float32),
            pltpu.SemaphoreType.DMA,
        ],
    )
    def sc_kernel(idx_hbm, pos_hbm, neg_hbm, table_hbm,
                  user_out, pos_out, neg_out,
                  idx_v, rows_v, uout_v, pidx_v, prow_v, nidx_v, nrow_v, sem):
        wid = lax.axis_index("s") * NC + lax.axis_index("c")
        base = wid * BPT

        def group_body(g, carry):
            off = (base + g * G) * L
            pltpu.sync_copy(idx_hbm.at[pl.ds(off, G * L)], idx_v)
            handles = []
            for c in range(NCH):
                handles.append(pltpu.async_copy(
                    table_hbm.at[idx_v.at[pl.ds(c * CHUNK, CHUNK)]],
                    rows_v.at[pl.ds(c * CHUNK, CHUNK), :],
                    sem))
            for h in handles:
                h.wait()
            for b in range(G):
                def row_body(l, acc):
                    return tuple(
                        acc[j] + rows_v[b * L + l, pl.ds(j * 16, 16)]
                        for j in range(NV))
                acc = lax.fori_loop(
                    0, L, row_body,
                    tuple(jnp.zeros((16,), jnp.float32) for _ in range(NV)))
                for j in range(NV):
                    uout_v[g * G + b, pl.ds(j * 16, 16)] = acc[j]
            return carry

        lax.fori_loop(0, NG, group_body, 0)
        pltpu.sync_copy(uout_v, user_out.at[pl.ds(base, BPT)])

        pltpu.sync_copy(pos_hbm.at[pl.ds(base, BPT)], pidx_v)
        pltpu.async_copy(table_hbm.at[pidx_v], prow_v, sem).wait()
        pltpu.sync_copy(prow_v, pos_out.at[pl.ds(base, BPT)])

        nbase = base * NEGK
        pltpu.sync_copy(neg_hbm.at[pl.ds(nbase, BPT * NEGK)], nidx_v)
        handles = []
        for c in range(NEGK):
            handles.append(pltpu.async_copy(
                table_hbm.at[nidx_v.at[pl.ds(c * BPT, BPT)]],
                nrow_v.at[pl.ds(c * BPT, BPT), :],
                sem))
        for h in handles:
            h.wait()
        pltpu.sync_copy(nrow_v, neg_out.at[pl.ds(nbase, BPT * NEGK)])

    return sc_kernel(idx_flat, pos, neg, table)


def _tc_head(user_sum, click_seq, table0, pos_info, neg_flat,
             W1, b1, W2, b2, B, L, NEGK, D):
    H1 = W1.shape[1]
    BLK = 512

    def body(us_ref, ck_ref, t0_ref, pos_ref, neg_ref,
             w1_ref, b1_ref, w2_ref, b2_ref, out_ref):
        z = jnp.sum((ck_ref[...] == 0).astype(jnp.float32), axis=1,
                    keepdims=True)
        um = (us_ref[...] - z * t0_ref[...]) * (1.0 / L)
        h = jnp.maximum(
            jnp.dot(um, w1_ref[...], preferred_element_type=jnp.float32)
            + b1_ref[...], 0.0)
        u = jnp.maximum(
            jnp.dot(h, w2_ref[...], preferred_element_type=jnp.float32)
            + b2_ref[...], 0.0)
        cols = [jnp.sum(u * pos_ref[...], axis=1, keepdims=True)]
        for j in range(NEGK):
            cols.append(jnp.sum(u * neg_ref[:, j * D:(j + 1) * D], axis=1,
                                keepdims=True))
        out_ref[...] = jnp.concatenate(cols, axis=1)

    return pl.pallas_call(
        body,
        grid=(B // BLK,),
        in_specs=[
            pl.BlockSpec((BLK, D), lambda i: (i, 0)),
            pl.BlockSpec((BLK, L), lambda i: (i, 0)),
            pl.BlockSpec((1, D), lambda i: (0, 0)),
            pl.BlockSpec((BLK, D), lambda i: (i, 0)),
            pl.BlockSpec((BLK, NEGK * D), lambda i: (i, 0)),
            pl.BlockSpec((D, H1), lambda i: (0, 0)),
            pl.BlockSpec((1, H1), lambda i: (0, 0)),
            pl.BlockSpec((H1, D), lambda i: (0, 0)),
            pl.BlockSpec((1, D), lambda i: (0, 0)),
        ],
        out_specs=pl.BlockSpec((BLK, 1 + NEGK), lambda i: (i, 0)),
        out_shape=jax.ShapeDtypeStruct((B, 1 + NEGK), jnp.float32),
    )(user_sum, click_seq, table0, pos_info, neg_flat, W1, b1, W2, b2)


def kernel(click_seq, pos_item, neg_item, table, W1, b1, W2, b2):
    B, L = click_seq.shape
    NEGK = neg_item.shape[1]
    D = table.shape[1]

    click_seq = click_seq.astype(jnp.int32)
    idx_flat = click_seq.reshape(-1)
    pos = pos_item.astype(jnp.int32)
    neg = neg_item.astype(jnp.int32).reshape(-1)

    user_sum, pos_info, neg_info = _sc_gather_sum(
        idx_flat, pos, neg, table, B, L, NEGK, D)

    table0 = lax.slice(table, (0, 0), (1, D))
    neg_flat = neg_info.reshape(B, NEGK * D)
    return _tc_head(user_sum, click_seq, table0, pos_info, neg_flat,
                    W1, b1.reshape(1, -1), W2, b2.reshape(1, -1),
                    B, L, NEGK, D)
